# baseline (speedup 1.0000x reference)
import functools
import jax
import jax.numpy as jnp
from jax import lax
from jax.experimental import pallas as pl
from jax.experimental.pallas import tpu as pltpu
from jax.experimental.pallas import tpu_sc as plsc

EPS = 1e-15
L = 16
G = 4

def _pre_body(x_ref, w_ref, x2_ref, xw_ref):
    xb = x_ref[...]
    x2_ref[...] = jnp.sum(xb * xb, axis=-1, keepdims=True)
    xw_ref[...] = jnp.sum(xb * w_ref[...], axis=-1, keepdims=True)


def _pre_call(xf, wrow, blk):
    n, d = xf.shape
    grid = n // blk
    return pl.pallas_call(
        _pre_body,
        grid=(grid,),
        in_specs=[
            pl.BlockSpec((blk, d), lambda i: (i, 0)),
            pl.BlockSpec((1, d), lambda i: (0, 0)),
        ],
        out_specs=[
            pl.BlockSpec((blk, 1), lambda i: (i, 0)),
            pl.BlockSpec((blk, 1), lambda i: (i, 0)),
        ],
        out_shape=[
            jax.ShapeDtypeStruct((n, 1), jnp.float32),
            jax.ShapeDtypeStruct((n, 1), jnp.float32),
        ],
    )(xf, wrow)


def _make_sc_kernel(n, d, k):
    e_chunk = G * k
    nchunk = n // G
    info = plsc.get_sparse_core_info()
    nc, ns = info.num_cores, info.num_subcores
    nw = nc * ns
    nci = (nchunk + nw - 1) // nw
    nvr = d // L

    mesh = plsc.VectorSubcoreMesh(core_axis_name="c", subcore_axis_name="s")

    @functools.partial(
        pl.kernel,
        mesh=mesh,
        compiler_params=pltpu.CompilerParams(needs_layout_passes=False),
        out_type=jax.ShapeDtypeStruct((n, d), jnp.float32),
        scratch_types=[
            pltpu.VMEM((nci * e_chunk,), jnp.int32),
            pltpu.VMEM((nci * e_chunk,), jnp.int32),
            pltpu.VMEM((e_chunk, d), jnp.float32),
            pltpu.VMEM((e_chunk, d), jnp.float32),
            pltpu.VMEM((e_chunk, d), jnp.float32),
            pltpu.VMEM((e_chunk, d), jnp.float32),
            pltpu.VMEM((n,), jnp.float32),
            pltpu.VMEM((n,), jnp.float32),
            pltpu.VMEM((e_chunk * L,), jnp.float32),
            pltpu.VMEM((e_chunk,), jnp.float32),
            pltpu.VMEM((e_chunk,), jnp.float32),
            pltpu.VMEM((L,), jnp.float32),
            pltpu.VMEM((G, d), jnp.float32),
            pltpu.VMEM((G, d), jnp.float32),
            pltpu.SemaphoreType.DMA,
            pltpu.SemaphoreType.DMA,
            pltpu.SemaphoreType.DMA,
            pltpu.SemaphoreType.DMA,
        ],
    )
    def sc_kernel(x_hbm, row_hbm, col_hbm, x2_hbm, xw_hbm, bvec_hbm, out_hbm,
                  idx_ar, idx_ac, rows_r0, rows_c0, rows_r1, rows_c1,
                  x2_v, xw_v, part_v, u_v, v_v, bvec_v, acc0, acc1,
                  sem0, sem1, osem0, osem1):
        wid = lax.axis_index("s") * nc + lax.axis_index("c")
        rows_rs = [rows_r0, rows_r1]
        rows_cs = [rows_c0, rows_c1]
        accs = [acc0, acc1]
        sems = [sem0, sem1]
        osems = [osem0, osem1]

        base_chunk = wid * nci
        nvalid = jnp.minimum(nchunk - base_chunk, nci)
        ebase0 = base_chunk * e_chunk
        stage0 = jnp.minimum(ebase0, n * k - nci * e_chunk)
        delta = ebase0 - stage0
        pltpu.async_copy(row_hbm.at[pl.ds(stage0, nci * e_chunk)], idx_ar,
                         osem0)
        pltpu.async_copy(col_hbm.at[pl.ds(stage0, nci * e_chunk)], idx_ac,
                         osem0)
        pltpu.async_copy(x2_hbm, x2_v, osem1)
        pltpu.async_copy(xw_hbm, xw_v, osem1)
        pltpu.async_copy(bvec_hbm, bvec_v, osem1)
        pltpu.make_async_copy(row_hbm.at[pl.ds(stage0, nci * e_chunk)],
                              idx_ar, osem0).wait()
        pltpu.make_async_copy(col_hbm.at[pl.ds(stage0, nci * e_chunk)],
                              idx_ac, osem0).wait()

        def issue(j, b):
            @pl.when(j < nvalid)
            def _():
                off = delta + j * e_chunk
                pltpu.async_copy(
                    x_hbm.at[idx_ar.at[pl.ds(off, e_chunk)]],
                    rows_rs[b], sems[b])
                pltpu.async_copy(
                    x_hbm.at[idx_ac.at[pl.ds(off, e_chunk)]],
                    rows_cs[b], sems[b])

        def compute(j, b):
            rows_r = rows_rs[b]
            rows_c = rows_cs[b]
            acc_v = accs[b]

            @pl.when(j < nvalid)
            def _():
                nbase = (base_chunk + j) * G
                off = delta + j * e_chunk
                pltpu.make_async_copy(
                    x_hbm.at[idx_ar.at[pl.ds(off, e_chunk)]],
                    rows_r, sems[b]).wait()
                pltpu.make_async_copy(
                    x_hbm.at[idx_ac.at[pl.ds(off, e_chunk)]],
                    rows_c, sems[b]).wait()

                @pl.when(j >= 2)
                def _():
                    pltpu.make_async_copy(
                        acc_v, out_hbm.at[pl.ds(nbase, G)], osems[b]).wait()

                def node_body(nn, c):
                    acc = tuple(jnp.zeros((L,), jnp.float32)
                                for _ in range(nvr))
                    for h in range(k // L):
                        gbase = h * L
                        for j in range(L):
                            e = nn * k + gbase + j
                            a0 = (rows_r[e, pl.ds(0, L)]
                                  * rows_c[e, pl.ds(0, L)])
                            a1 = (rows_r[e, pl.ds(L, L)]
                                  * rows_c[e, pl.ds(L, L)])
                            for v in range(2, nvr, 2):
                                a0 = a0 + (rows_r[e, pl.ds(v * L, L)]
                                           * rows_c[e, pl.ds(v * L, L)])
                                a1 = a1 + (rows_r[e, pl.ds((v + 1) * L, L)]
                                           * rows_c[e, pl.ds((v + 1) * L, L)])
                            part_v[pl.ds((gbase + j) * L, L)] = a0 + a1
                        iota = lax.iota(jnp.int32, L)
                        terms = [plsc.load_gather(
                                     part_v, [gbase * L + iota * L + j2])
                                 for j2 in range(L)]
                        while len(terms) > 1:
                            nxt = [terms[i] + terms[i + 1]
                                   for i in range(0, len(terms) - 1, 2)]
                            if len(terms) % 2:
                                nxt.append(terms[-1])
                            terms = nxt
                        pq = terms[0]
                        ebase = off + nn * k + gbase
                        ir = idx_ar[pl.ds(ebase, L)]
                        ic = idx_ac[pl.ds(ebase, L)]
                        x2r = plsc.load_gather(x2_v, [ir])
                        y2 = plsc.load_gather(x2_v, [ic])
                        pw = plsc.load_gather(xw_v, [ir])
                        qw = plsc.load_gather(xw_v, [ic])
                        den = jnp.maximum(1.0 - 2.0 * pq + x2r * y2, EPS)
                        alpha = -(1.0 - 2.0 * pq + y2) / den
                        beta = (1.0 - x2r) / den
                        sn2 = (alpha * alpha * x2r + 2.0 * alpha * beta * pq
                               + beta * beta * y2 + EPS)
                        gpoly = 1.0 / 15.0
                        for coef in (1.0 / 13.0, 1.0 / 11.0, 1.0 / 9.0,
                                     1.0 / 7.0, 1.0 / 5.0, 1.0 / 3.0, 1.0):
                            gpoly = gpoly * sn2 + coef
                        s = jnp.maximum(1.0 - x2r, EPS) * gpoly
                        logit = s * (alpha * pw + beta * qw) + bvec
                        aw = 1.0 / (1.0 + jnp.exp(-logit))
                        uvec = aw * s * alpha
                        vvec = aw * s * beta
                        for j in range(L):
                            e = nn * k + gbase + j
                            iv = jnp.full((L,), j, jnp.int32)
                            uu = uvec.at[iv].get(mode="promise_in_bounds")
                            vv = vvec.at[iv].get(mode="promise_in_bounds")
                            acc = tuple(
                                acc[v] + uu * rows_r[e, pl.ds(v * L, L)]
                                + vv * rows_c[e, pl.ds(v * L, L)]
                                for v in range(nvr))
                    for v in range(nvr):
                        acc_v[nn, pl.ds(v * L, L)] = acc[v]
                    return c

                lax.fori_loop(0, G, node_body, 0)

                pltpu.async_copy(acc_v, out_hbm.at[pl.ds(nbase, G)], osems[b])

        issue(0, 0)
        issue(1, 1)
        pltpu.make_async_copy(x2_hbm, x2_v, osem1).wait()
        pltpu.make_async_copy(xw_hbm, xw_v, osem1).wait()
        pltpu.make_async_copy(bvec_hbm, bvec_v, osem1).wait()
        bvec = bvec_v[...]

        def pair_body(i2, carry):
            for b in range(2):
                j = i2 * 2 + b
                compute(j, b)
                issue(j + 2, b)
            return carry

        lax.fori_loop(0, (nci + 1) // 2, pair_body, 0)

        for b in range(2):
            @pl.when(nvalid > b)
            def _():
                pltpu.make_async_copy(
                    accs[b], out_hbm.at[pl.ds(0, G)], osems[b]).wait()

    return sc_kernel


def _post_body(sup_ref, x_ref, w1_ref, b1_ref, w2_ref, b2_ref, out_ref):
    s = sup_ref[...]
    h1 = jnp.dot(s, w1_ref[...], preferred_element_type=jnp.float32)
    h1 = h1 + b1_ref[...]
    h = h1 / (1.0 + jnp.exp(-h1))
    st = jnp.dot(h, w2_ref[...], preferred_element_type=jnp.float32)
    st = st + b2_ref[...]
    xb = x_ref[...]
    x2 = jnp.sum(xb * xb, axis=-1, keepdims=True)
    un = jnp.sqrt(jnp.sum(st * st, axis=-1, keepdims=True) + EPS)
    lam = 2.0 / jnp.maximum(1.0 - x2, EPS)
    second = jnp.tanh(0.5 * lam * un) * st / un
    s2 = jnp.sum(second * second, axis=-1, keepdims=True)
    xs = jnp.sum(xb * second, axis=-1, keepdims=True)
    num = (1.0 + 2.0 * xs + s2) * xb + (1.0 - x2) * second
    dn = jnp.maximum(1.0 + 2.0 * xs + x2 * s2, EPS)
    o = num / dn
    on = jnp.sqrt(jnp.sum(o * o, axis=-1, keepdims=True) + EPS)
    mx = 1.0 - 1e-3
    out_ref[...] = jnp.where(on > mx, o / on * mx, o)


def _post_call(sup, xf, w1b, b1, w2, b2, blk):
    n, d = xf.shape
    grid = n // blk
    full = lambda i: (0, 0)
    return pl.pallas_call(
        _post_body,
        grid=(grid,),
        in_specs=[
            pl.BlockSpec((blk, d), lambda i: (i, 0)),
            pl.BlockSpec((blk, d), lambda i: (i, 0)),
            pl.BlockSpec((d, d), full),
            pl.BlockSpec((1, d), full),
            pl.BlockSpec((d, d), full),
            pl.BlockSpec((1, d), full),
        ],
        out_specs=pl.BlockSpec((blk, d), lambda i: (i, 0)),
        out_shape=jax.ShapeDtypeStruct((n, d), jnp.float32),
    )(sup, xf, w1b, b1, w2, b2)


def kernel(x, distances, edges, node_mask, edge_mask, att_W, att_b, W1, b1,
           W2, b2):
    b, n, d = x.shape
    k = edges.shape[2]
    xf = x.reshape(n, d)
    wrow = att_W[:d, 0].reshape(1, d)
    x2, xw = _pre_call(xf, wrow, blk=2000)
    rows = edges[0].reshape(n * k)
    cols = edges[1].reshape(n * k)
    bvec = jnp.broadcast_to(att_b.astype(jnp.float32), (L,))
    sup = _make_sc_kernel(n, d, k)(
        xf, rows, cols, x2.reshape(n), xw.reshape(n), bvec)
    out = _post_call(sup, xf, W1[d:], b1.reshape(1, d), W2, b2.reshape(1, d),
                     blk=2000)
    return out.reshape(b, n, d)


# speedup vs baseline: 3.8210x; 3.8210x over previous
import functools
import jax
import jax.numpy as jnp
from jax import lax
from jax.experimental import pallas as pl
from jax.experimental.pallas import tpu as pltpu
from jax.experimental.pallas import tpu_sc as plsc

EPS = 1e-15
L = 16
G = 4

def _pre_body(x_ref, w_ref, x2_ref, xw_ref):
    xb = x_ref[...]
    x2_ref[...] = jnp.sum(xb * xb, axis=-1, keepdims=True)
    xw_ref[...] = jnp.sum(xb * w_ref[...], axis=-1, keepdims=True)


def _pre_call(xf, wrow, blk):
    n, d = xf.shape
    grid = n // blk
    return pl.pallas_call(
        _pre_body,
        grid=(grid,),
        in_specs=[
            pl.BlockSpec((blk, d), lambda i: (i, 0)),
            pl.BlockSpec((1, d), lambda i: (0, 0)),
        ],
        out_specs=[
            pl.BlockSpec((blk, 1), lambda i: (i, 0)),
            pl.BlockSpec((blk, 1), lambda i: (i, 0)),
        ],
        out_shape=[
            jax.ShapeDtypeStruct((n, 1), jnp.float32),
            jax.ShapeDtypeStruct((n, 1), jnp.float32),
        ],
    )(xf, wrow)


def _make_sc_kernel(n, d, k):
    e_chunk = G * k
    nchunk = n // G
    info = plsc.get_sparse_core_info()
    nc, ns = info.num_cores, info.num_subcores
    nw = nc * ns
    nci = (nchunk + nw - 1) // nw
    nvr = d // L

    mesh = plsc.VectorSubcoreMesh(core_axis_name="c", subcore_axis_name="s")

    @functools.partial(
        pl.kernel,
        mesh=mesh,
        compiler_params=pltpu.CompilerParams(needs_layout_passes=False),
        out_type=jax.ShapeDtypeStruct((n, d), jnp.float32),
        scratch_types=[
            pltpu.VMEM((nci * e_chunk,), jnp.int32),
            pltpu.VMEM((nci * e_chunk,), jnp.int32),
            pltpu.VMEM((e_chunk, d), jnp.float32),
            pltpu.VMEM((e_chunk, d), jnp.float32),
            pltpu.VMEM((e_chunk, d), jnp.float32),
            pltpu.VMEM((e_chunk, d), jnp.float32),
            pltpu.VMEM((n,), jnp.float32),
            pltpu.VMEM((n,), jnp.float32),
            pltpu.VMEM((e_chunk * L,), jnp.float32),
            pltpu.VMEM((e_chunk,), jnp.float32),
            pltpu.VMEM((e_chunk,), jnp.float32),
            pltpu.VMEM((L,), jnp.float32),
            pltpu.VMEM((G, d), jnp.float32),
            pltpu.VMEM((G, d), jnp.float32),
            pltpu.SemaphoreType.DMA,
            pltpu.SemaphoreType.DMA,
            pltpu.SemaphoreType.DMA,
            pltpu.SemaphoreType.DMA,
        ],
    )
    def sc_kernel(x_hbm, row_hbm, col_hbm, x2_hbm, xw_hbm, bvec_hbm, out_hbm,
                  idx_ar, idx_ac, rows_r0, rows_c0, rows_r1, rows_c1,
                  x2_v, xw_v, part_v, u_v, v_v, bvec_v, acc0, acc1,
                  sem0, sem1, osem0, osem1):
        wid = lax.axis_index("s") * nc + lax.axis_index("c")
        rows_rs = [rows_r0, rows_r1]
        rows_cs = [rows_c0, rows_c1]
        accs = [acc0, acc1]
        sems = [sem0, sem1]
        osems = [osem0, osem1]

        base_chunk = wid * nci
        nvalid = jnp.minimum(nchunk - base_chunk, nci)
        ebase0 = base_chunk * e_chunk
        stage0 = jnp.minimum(ebase0, n * k - nci * e_chunk)
        delta = ebase0 - stage0
        pltpu.async_copy(row_hbm.at[pl.ds(stage0, nci * e_chunk)], idx_ar,
                         osem0)
        pltpu.async_copy(col_hbm.at[pl.ds(stage0, nci * e_chunk)], idx_ac,
                         osem0)
        pltpu.async_copy(x2_hbm, x2_v, osem1)
        pltpu.async_copy(xw_hbm, xw_v, osem1)
        pltpu.async_copy(bvec_hbm, bvec_v, osem1)
        pltpu.make_async_copy(row_hbm.at[pl.ds(stage0, nci * e_chunk)],
                              idx_ar, osem0).wait()
        pltpu.make_async_copy(col_hbm.at[pl.ds(stage0, nci * e_chunk)],
                              idx_ac, osem0).wait()

        def issue(j, b):
            @pl.when(j < nvalid)
            def _():
                off = delta + j * e_chunk
                pltpu.async_copy(
                    x_hbm.at[idx_ar.at[pl.ds(off, e_chunk)]],
                    rows_rs[b], sems[b])
                pltpu.async_copy(
                    x_hbm.at[idx_ac.at[pl.ds(off, e_chunk)]],
                    rows_cs[b], sems[b])

        def compute(j, b):
            rows_r = rows_rs[b]
            rows_c = rows_cs[b]
            acc_v = accs[b]

            @pl.when(j < nvalid)
            def _():
                nbase = (base_chunk + j) * G
                off = delta + j * e_chunk
                pltpu.make_async_copy(
                    x_hbm.at[idx_ar.at[pl.ds(off, e_chunk)]],
                    rows_r, sems[b]).wait()
                pltpu.make_async_copy(
                    x_hbm.at[idx_ac.at[pl.ds(off, e_chunk)]],
                    rows_c, sems[b]).wait()

                @pl.when(j >= 2)
                def _():
                    pltpu.make_async_copy(
                        acc_v, out_hbm.at[pl.ds(nbase, G)], osems[b]).wait()

                @plsc.parallel_loop(0, e_chunk, unroll=4)
                def _(e):
                    acc0 = rows_r[e, pl.ds(0, L)] * rows_c[e, pl.ds(0, L)]
                    acc1 = rows_r[e, pl.ds(L, L)] * rows_c[e, pl.ds(L, L)]
                    for v in range(2, nvr, 2):
                        acc0 = acc0 + (rows_r[e, pl.ds(v * L, L)]
                                       * rows_c[e, pl.ds(v * L, L)])
                        acc1 = acc1 + (rows_r[e, pl.ds((v + 1) * L, L)]
                                       * rows_c[e, pl.ds((v + 1) * L, L)])
                    part_v[pl.ds(e * L, L)] = acc0 + acc1

                @plsc.parallel_loop(0, e_chunk // L, unroll=2)
                def _(gi):
                    base = gi * L
                    evec = (base + lax.iota(jnp.int32, L)) * L
                    terms = [plsc.load_gather(part_v, [evec + j2])
                             for j2 in range(L)]
                    while len(terms) > 1:
                        nxt = [terms[i] + terms[i + 1]
                               for i in range(0, len(terms) - 1, 2)]
                        if len(terms) % 2:
                            nxt.append(terms[-1])
                        terms = nxt
                    pq = terms[0]
                    ir = idx_ar[pl.ds(off + base, L)]
                    ic = idx_ac[pl.ds(off + base, L)]
                    x2r = plsc.load_gather(x2_v, [ir])
                    y2 = plsc.load_gather(x2_v, [ic])
                    pw = plsc.load_gather(xw_v, [ir])
                    qw = plsc.load_gather(xw_v, [ic])
                    den = jnp.maximum(1.0 - 2.0 * pq + x2r * y2, EPS)
                    alpha = -(1.0 - 2.0 * pq + y2) / den
                    beta = (1.0 - x2r) / den
                    sn2 = (alpha * alpha * x2r + 2.0 * alpha * beta * pq
                           + beta * beta * y2 + EPS)
                    gpoly = 1.0 / 15.0
                    for coef in (1.0 / 13.0, 1.0 / 11.0, 1.0 / 9.0, 1.0 / 7.0,
                                 1.0 / 5.0, 1.0 / 3.0, 1.0):
                        gpoly = gpoly * sn2 + coef
                    s = jnp.maximum(1.0 - x2r, EPS) * gpoly
                    logit = s * (alpha * pw + beta * qw) + bvec
                    a = 1.0 / (1.0 + jnp.exp(-logit))
                    u_v[pl.ds(base, L)] = a * s * alpha
                    v_v[pl.ds(base, L)] = a * s * beta

                for nn in range(G):
                    acc = tuple(jnp.zeros((L,), jnp.float32)
                                for _ in range(nvr))
                    for h in range(k // L):
                        ubase = nn * k + h * L
                        uvec = u_v[pl.ds(ubase, L)]
                        vvec = v_v[pl.ds(ubase, L)]

                        @plsc.parallel_loop(0, L, unroll=2, carry=acc)
                        def acc(kk, carry, uvec=uvec, vvec=vvec,
                                ubase=ubase):
                            e = ubase + kk
                            iv = jnp.full((L,), 0, jnp.int32) + kk
                            uu = uvec.at[iv].get(mode="promise_in_bounds")
                            vv = vvec.at[iv].get(mode="promise_in_bounds")
                            return tuple(
                                carry[v] + uu * rows_r[e, pl.ds(v * L, L)]
                                + vv * rows_c[e, pl.ds(v * L, L)]
                                for v in range(nvr))

                    for v in range(nvr):
                        acc_v[nn, pl.ds(v * L, L)] = acc[v]

                pltpu.async_copy(acc_v, out_hbm.at[pl.ds(nbase, G)], osems[b])

        issue(0, 0)
        issue(1, 1)
        pltpu.make_async_copy(x2_hbm, x2_v, osem1).wait()
        pltpu.make_async_copy(xw_hbm, xw_v, osem1).wait()
        pltpu.make_async_copy(bvec_hbm, bvec_v, osem1).wait()
        bvec = bvec_v[...]

        def pair_body(i2, carry):
            for b in range(2):
                j = i2 * 2 + b
                compute(j, b)
                issue(j + 2, b)
            return carry

        lax.fori_loop(0, (nci + 1) // 2, pair_body, 0)

        for b in range(2):
            @pl.when(nvalid > b)
            def _():
                pltpu.make_async_copy(
                    accs[b], out_hbm.at[pl.ds(0, G)], osems[b]).wait()

    return sc_kernel


def _post_body(sup_ref, x_ref, w1_ref, b1_ref, w2_ref, b2_ref, out_ref):
    s = sup_ref[...]
    h1 = jnp.dot(s, w1_ref[...], preferred_element_type=jnp.float32)
    h1 = h1 + b1_ref[...]
    h = h1 / (1.0 + jnp.exp(-h1))
    st = jnp.dot(h, w2_ref[...], preferred_element_type=jnp.float32)
    st = st + b2_ref[...]
    xb = x_ref[...]
    x2 = jnp.sum(xb * xb, axis=-1, keepdims=True)
    un = jnp.sqrt(jnp.sum(st * st, axis=-1, keepdims=True) + EPS)
    lam = 2.0 / jnp.maximum(1.0 - x2, EPS)
    second = jnp.tanh(0.5 * lam * un) * st / un
    s2 = jnp.sum(second * second, axis=-1, keepdims=True)
    xs = jnp.sum(xb * second, axis=-1, keepdims=True)
    num = (1.0 + 2.0 * xs + s2) * xb + (1.0 - x2) * second
    dn = jnp.maximum(1.0 + 2.0 * xs + x2 * s2, EPS)
    o = num / dn
    on = jnp.sqrt(jnp.sum(o * o, axis=-1, keepdims=True) + EPS)
    mx = 1.0 - 1e-3
    out_ref[...] = jnp.where(on > mx, o / on * mx, o)


def _post_call(sup, xf, w1b, b1, w2, b2, blk):
    n, d = xf.shape
    grid = n // blk
    full = lambda i: (0, 0)
    return pl.pallas_call(
        _post_body,
        grid=(grid,),
        in_specs=[
            pl.BlockSpec((blk, d), lambda i: (i, 0)),
            pl.BlockSpec((blk, d), lambda i: (i, 0)),
            pl.BlockSpec((d, d), full),
            pl.BlockSpec((1, d), full),
            pl.BlockSpec((d, d), full),
            pl.BlockSpec((1, d), full),
        ],
        out_specs=pl.BlockSpec((blk, d), lambda i: (i, 0)),
        out_shape=jax.ShapeDtypeStruct((n, d), jnp.float32),
    )(sup, xf, w1b, b1, w2, b2)


def kernel(x, distances, edges, node_mask, edge_mask, att_W, att_b, W1, b1,
           W2, b2):
    b, n, d = x.shape
    k = edges.shape[2]
    xf = x.reshape(n, d)
    wrow = att_W[:d, 0].reshape(1, d)
    x2, xw = _pre_call(xf, wrow, blk=2000)
    rows = edges[0].reshape(n * k)
    cols = edges[1].reshape(n * k)
    bvec = jnp.broadcast_to(att_b.astype(jnp.float32), (L,))
    sup = _make_sc_kernel(n, d, k)(
        xf, rows, cols, x2.reshape(n), xw.reshape(n), bvec)
    out = _post_call(sup, xf, W1[d:], b1.reshape(1, d), W2, b2.reshape(1, d),
                     blk=2000)
    return out.reshape(b, n, d)


# speedup vs baseline: 3.8491x; 1.0074x over previous
import functools
import jax
import jax.numpy as jnp
from jax import lax
from jax.experimental import pallas as pl
from jax.experimental.pallas import tpu as pltpu
from jax.experimental.pallas import tpu_sc as plsc

EPS = 1e-15
L = 16
G = 4

def _pre_body(x_ref, w_ref, x2_ref, xw_ref):
    xb = x_ref[...]
    x2_ref[...] = jnp.sum(xb * xb, axis=-1, keepdims=True)
    xw_ref[...] = jnp.sum(xb * w_ref[...], axis=-1, keepdims=True)


def _pre_call(xf, wrow, blk):
    n, d = xf.shape
    grid = n // blk
    return pl.pallas_call(
        _pre_body,
        grid=(grid,),
        in_specs=[
            pl.BlockSpec((blk, d), lambda i: (i, 0)),
            pl.BlockSpec((1, d), lambda i: (0, 0)),
        ],
        out_specs=[
            pl.BlockSpec((blk, 1), lambda i: (i, 0)),
            pl.BlockSpec((blk, 1), lambda i: (i, 0)),
        ],
        out_shape=[
            jax.ShapeDtypeStruct((n, 1), jnp.float32),
            jax.ShapeDtypeStruct((n, 1), jnp.float32),
        ],
    )(xf, wrow)


def _make_sc_kernel(n, d, k):
    e_chunk = G * k
    nchunk = n // G
    info = plsc.get_sparse_core_info()
    nc, ns = info.num_cores, info.num_subcores
    nw = nc * ns
    nci = (nchunk + nw - 1) // nw
    nvr = d // L

    mesh = plsc.VectorSubcoreMesh(core_axis_name="c", subcore_axis_name="s")

    @functools.partial(
        pl.kernel,
        mesh=mesh,
        compiler_params=pltpu.CompilerParams(needs_layout_passes=False),
        out_type=jax.ShapeDtypeStruct((n, d), jnp.float32),
        scratch_types=[
            pltpu.VMEM((nci * e_chunk,), jnp.int32),
            pltpu.VMEM((nci * e_chunk,), jnp.int32),
            pltpu.VMEM((e_chunk, d), jnp.float32),
            pltpu.VMEM((e_chunk, d), jnp.float32),
            pltpu.VMEM((e_chunk, d), jnp.float32),
            pltpu.VMEM((e_chunk, d), jnp.float32),
            pltpu.VMEM((n,), jnp.float32),
            pltpu.VMEM((n,), jnp.float32),
            pltpu.VMEM((e_chunk * L,), jnp.float32),
            pltpu.VMEM((e_chunk,), jnp.float32),
            pltpu.VMEM((e_chunk,), jnp.float32),
            pltpu.VMEM((L,), jnp.float32),
            pltpu.VMEM((G, d), jnp.float32),
            pltpu.VMEM((G, d), jnp.float32),
            pltpu.SemaphoreType.DMA,
            pltpu.SemaphoreType.DMA,
            pltpu.SemaphoreType.DMA,
            pltpu.SemaphoreType.DMA,
        ],
    )
    def sc_kernel(x_hbm, row_hbm, col_hbm, x2_hbm, xw_hbm, bvec_hbm, out_hbm,
                  idx_ar, idx_ac, rows_r0, rows_c0, rows_r1, rows_c1,
                  x2_v, xw_v, part_v, u_v, v_v, bvec_v, acc0, acc1,
                  sem0, sem1, osem0, osem1):
        wid = lax.axis_index("s") * nc + lax.axis_index("c")
        rows_rs = [rows_r0, rows_r1]
        rows_cs = [rows_c0, rows_c1]
        accs = [acc0, acc1]
        sems = [sem0, sem1]
        osems = [osem0, osem1]

        base_chunk = wid * nci
        nvalid = jnp.minimum(nchunk - base_chunk, nci)
        ebase0 = base_chunk * e_chunk
        stage0 = jnp.minimum(ebase0, n * k - nci * e_chunk)
        delta = ebase0 - stage0
        pltpu.async_copy(row_hbm.at[pl.ds(stage0, nci * e_chunk)], idx_ar,
                         osem0)
        pltpu.async_copy(col_hbm.at[pl.ds(stage0, nci * e_chunk)], idx_ac,
                         osem0)
        pltpu.async_copy(x2_hbm, x2_v, osem1)
        pltpu.async_copy(xw_hbm, xw_v, osem1)
        pltpu.async_copy(bvec_hbm, bvec_v, osem1)
        pltpu.make_async_copy(row_hbm.at[pl.ds(stage0, nci * e_chunk)],
                              idx_ar, osem0).wait()
        pltpu.make_async_copy(col_hbm.at[pl.ds(stage0, nci * e_chunk)],
                              idx_ac, osem0).wait()

        def issue(j, b):
            @pl.when(j < nvalid)
            def _():
                off = delta + j * e_chunk
                pltpu.async_copy(
                    x_hbm.at[idx_ar.at[pl.ds(off, e_chunk)]],
                    rows_rs[b], sems[b])
                pltpu.async_copy(
                    x_hbm.at[idx_ac.at[pl.ds(off, e_chunk)]],
                    rows_cs[b], sems[b])

        def compute(j, b):
            rows_r = rows_rs[b]
            rows_c = rows_cs[b]
            acc_v = accs[b]

            @pl.when(j < nvalid)
            def _():
                nbase = (base_chunk + j) * G
                off = delta + j * e_chunk
                pltpu.make_async_copy(
                    x_hbm.at[idx_ar.at[pl.ds(off, e_chunk)]],
                    rows_r, sems[b]).wait()
                pltpu.make_async_copy(
                    x_hbm.at[idx_ac.at[pl.ds(off, e_chunk)]],
                    rows_c, sems[b]).wait()

                @pl.when(j >= 2)
                def _():
                    pltpu.make_async_copy(
                        acc_v, out_hbm.at[pl.ds(nbase, G)], osems[b]).wait()

                @plsc.parallel_loop(0, e_chunk, unroll=2)
                def _(e):
                    acc0 = rows_r[e, pl.ds(0, L)] * rows_c[e, pl.ds(0, L)]
                    acc1 = rows_r[e, pl.ds(L, L)] * rows_c[e, pl.ds(L, L)]
                    for v in range(2, nvr, 2):
                        acc0 = acc0 + (rows_r[e, pl.ds(v * L, L)]
                                       * rows_c[e, pl.ds(v * L, L)])
                        acc1 = acc1 + (rows_r[e, pl.ds((v + 1) * L, L)]
                                       * rows_c[e, pl.ds((v + 1) * L, L)])
                    part_v[pl.ds(e * L, L)] = acc0 + acc1

                @plsc.parallel_loop(0, e_chunk // L, unroll=2)
                def _(gi):
                    base = gi * L
                    evec = (base + lax.iota(jnp.int32, L)) * L
                    terms = [plsc.load_gather(part_v, [evec + j2])
                             for j2 in range(L)]
                    while len(terms) > 1:
                        nxt = [terms[i] + terms[i + 1]
                               for i in range(0, len(terms) - 1, 2)]
                        if len(terms) % 2:
                            nxt.append(terms[-1])
                        terms = nxt
                    pq = terms[0]
                    ir = idx_ar[pl.ds(off + base, L)]
                    ic = idx_ac[pl.ds(off + base, L)]
                    x2r = plsc.load_gather(x2_v, [ir])
                    y2 = plsc.load_gather(x2_v, [ic])
                    pw = plsc.load_gather(xw_v, [ir])
                    qw = plsc.load_gather(xw_v, [ic])
                    den = jnp.maximum(1.0 - 2.0 * pq + x2r * y2, EPS)
                    alpha = -(1.0 - 2.0 * pq + y2) / den
                    beta = (1.0 - x2r) / den
                    sn2 = (alpha * alpha * x2r + 2.0 * alpha * beta * pq
                           + beta * beta * y2 + EPS)
                    gpoly = 1.0 / 15.0
                    for coef in (1.0 / 13.0, 1.0 / 11.0, 1.0 / 9.0, 1.0 / 7.0,
                                 1.0 / 5.0, 1.0 / 3.0, 1.0):
                        gpoly = gpoly * sn2 + coef
                    s = jnp.maximum(1.0 - x2r, EPS) * gpoly
                    logit = s * (alpha * pw + beta * qw) + bvec
                    a = 1.0 / (1.0 + jnp.exp(-logit))
                    u_v[pl.ds(base, L)] = a * s * alpha
                    v_v[pl.ds(base, L)] = a * s * beta

                for nn in range(G):
                    acc = tuple(jnp.zeros((L,), jnp.float32)
                                for _ in range(nvr))
                    for h in range(k // L):
                        ubase = nn * k + h * L
                        uvec = u_v[pl.ds(ubase, L)]
                        vvec = v_v[pl.ds(ubase, L)]

                        @plsc.parallel_loop(0, L, unroll=2, carry=acc)
                        def acc(kk, carry, uvec=uvec, vvec=vvec,
                                ubase=ubase):
                            e = ubase + kk
                            iv = jnp.full((L,), 0, jnp.int32) + kk
                            uu = uvec.at[iv].get(mode="promise_in_bounds")
                            vv = vvec.at[iv].get(mode="promise_in_bounds")
                            return tuple(
                                carry[v] + uu * rows_r[e, pl.ds(v * L, L)]
                                + vv * rows_c[e, pl.ds(v * L, L)]
                                for v in range(nvr))

                    for v in range(nvr):
                        acc_v[nn, pl.ds(v * L, L)] = acc[v]

                pltpu.async_copy(acc_v, out_hbm.at[pl.ds(nbase, G)], osems[b])

        issue(0, 0)
        issue(1, 1)
        pltpu.make_async_copy(x2_hbm, x2_v, osem1).wait()
        pltpu.make_async_copy(xw_hbm, xw_v, osem1).wait()
        pltpu.make_async_copy(bvec_hbm, bvec_v, osem1).wait()
        bvec = bvec_v[...]

        def pair_body(i2, carry):
            for b in range(2):
                j = i2 * 2 + b
                compute(j, b)
                issue(j + 2, b)
            return carry

        lax.fori_loop(0, (nci + 1) // 2, pair_body, 0)

        for b in range(2):
            @pl.when(nvalid > b)
            def _():
                pltpu.make_async_copy(
                    accs[b], out_hbm.at[pl.ds(0, G)], osems[b]).wait()

    return sc_kernel


def _post_body(sup_ref, x_ref, w1_ref, b1_ref, w2_ref, b2_ref, out_ref):
    s = sup_ref[...]
    h1 = jnp.dot(s, w1_ref[...], preferred_element_type=jnp.float32)
    h1 = h1 + b1_ref[...]
    h = h1 / (1.0 + jnp.exp(-h1))
    st = jnp.dot(h, w2_ref[...], preferred_element_type=jnp.float32)
    st = st + b2_ref[...]
    xb = x_ref[...]
    x2 = jnp.sum(xb * xb, axis=-1, keepdims=True)
    un = jnp.sqrt(jnp.sum(st * st, axis=-1, keepdims=True) + EPS)
    lam = 2.0 / jnp.maximum(1.0 - x2, EPS)
    second = jnp.tanh(0.5 * lam * un) * st / un
    s2 = jnp.sum(second * second, axis=-1, keepdims=True)
    xs = jnp.sum(xb * second, axis=-1, keepdims=True)
    num = (1.0 + 2.0 * xs + s2) * xb + (1.0 - x2) * second
    dn = jnp.maximum(1.0 + 2.0 * xs + x2 * s2, EPS)
    o = num / dn
    on = jnp.sqrt(jnp.sum(o * o, axis=-1, keepdims=True) + EPS)
    mx = 1.0 - 1e-3
    out_ref[...] = jnp.where(on > mx, o / on * mx, o)


def _post_call(sup, xf, w1b, b1, w2, b2, blk):
    n, d = xf.shape
    grid = n // blk
    full = lambda i: (0, 0)
    return pl.pallas_call(
        _post_body,
        grid=(grid,),
        in_specs=[
            pl.BlockSpec((blk, d), lambda i: (i, 0)),
            pl.BlockSpec((blk, d), lambda i: (i, 0)),
            pl.BlockSpec((d, d), full),
            pl.BlockSpec((1, d), full),
            pl.BlockSpec((d, d), full),
            pl.BlockSpec((1, d), full),
        ],
        out_specs=pl.BlockSpec((blk, d), lambda i: (i, 0)),
        out_shape=jax.ShapeDtypeStruct((n, d), jnp.float32),
    )(sup, xf, w1b, b1, w2, b2)


def kernel(x, distances, edges, node_mask, edge_mask, att_W, att_b, W1, b1,
           W2, b2):
    b, n, d = x.shape
    k = edges.shape[2]
    xf = x.reshape(n, d)
    wrow = att_W[:d, 0].reshape(1, d)
    x2, xw = _pre_call(xf, wrow, blk=2000)
    rows = edges[0].reshape(n * k)
    cols = edges[1].reshape(n * k)
    bvec = jnp.broadcast_to(att_b.astype(jnp.float32), (L,))
    sup = _make_sc_kernel(n, d, k)(
        xf, rows, cols, x2.reshape(n), xw.reshape(n), bvec)
    out = _post_call(sup, xf, W1[d:], b1.reshape(1, d), W2, b2.reshape(1, d),
                     blk=2000)
    return out.reshape(b, n, d)


# speedup vs baseline: 4.1282x; 1.0725x over previous
import functools
import jax
import jax.numpy as jnp
from jax import lax
from jax.experimental import pallas as pl
from jax.experimental.pallas import tpu as pltpu
from jax.experimental.pallas import tpu_sc as plsc

EPS = 1e-15
L = 16
G = 4

def _pre_body(x_ref, w_ref, x2_ref, xw_ref):
    xb = x_ref[...]
    x2_ref[...] = jnp.sum(xb * xb, axis=-1, keepdims=True)
    xw_ref[...] = jnp.sum(xb * w_ref[...], axis=-1, keepdims=True)


def _pre_call(xf, wrow, blk):
    n, d = xf.shape
    grid = n // blk
    return pl.pallas_call(
        _pre_body,
        grid=(grid,),
        in_specs=[
            pl.BlockSpec((blk, d), lambda i: (i, 0)),
            pl.BlockSpec((1, d), lambda i: (0, 0)),
        ],
        out_specs=[
            pl.BlockSpec((blk, 1), lambda i: (i, 0)),
            pl.BlockSpec((blk, 1), lambda i: (i, 0)),
        ],
        out_shape=[
            jax.ShapeDtypeStruct((n, 1), jnp.float32),
            jax.ShapeDtypeStruct((n, 1), jnp.float32),
        ],
    )(xf, wrow)


def _make_sc_kernel(n, d, k):
    e_chunk = G * k
    nchunk = n // G
    info = plsc.get_sparse_core_info()
    nc, ns = info.num_cores, info.num_subcores
    nw = nc * ns
    nci = (nchunk + nw - 1) // nw
    nvr = d // L

    mesh = plsc.VectorSubcoreMesh(core_axis_name="c", subcore_axis_name="s")

    @functools.partial(
        pl.kernel,
        mesh=mesh,
        compiler_params=pltpu.CompilerParams(needs_layout_passes=False),
        out_type=jax.ShapeDtypeStruct((n, d), jnp.float32),
        scratch_types=[
            pltpu.VMEM((nci * e_chunk,), jnp.int32),
            pltpu.VMEM((nci * e_chunk,), jnp.int32),
            pltpu.VMEM((e_chunk, d), jnp.float32),
            pltpu.VMEM((e_chunk, d), jnp.float32),
            pltpu.VMEM((e_chunk, d), jnp.float32),
            pltpu.VMEM((e_chunk, d), jnp.float32),
            pltpu.VMEM((n,), jnp.float32),
            pltpu.VMEM((n,), jnp.float32),
            pltpu.VMEM((e_chunk * L,), jnp.float32),
            pltpu.VMEM((e_chunk,), jnp.float32),
            pltpu.VMEM((e_chunk,), jnp.float32),
            pltpu.VMEM((L,), jnp.float32),
            pltpu.VMEM((G, d), jnp.float32),
            pltpu.VMEM((G, d), jnp.float32),
            pltpu.SemaphoreType.DMA,
            pltpu.SemaphoreType.DMA,
            pltpu.SemaphoreType.DMA,
            pltpu.SemaphoreType.DMA,
        ],
    )
    def sc_kernel(x_hbm, row_hbm, col_hbm, x2_hbm, xw_hbm, bvec_hbm, out_hbm,
                  idx_ar, idx_ac, rows_r0, rows_c0, rows_r1, rows_c1,
                  x2_v, xw_v, part_v, u_v, v_v, bvec_v, acc0, acc1,
                  sem0, sem1, osem0, osem1):
        wid = lax.axis_index("s") * nc + lax.axis_index("c")
        rows_rs = [rows_r0, rows_r1]
        rows_cs = [rows_c0, rows_c1]
        accs = [acc0, acc1]
        sems = [sem0, sem1]
        osems = [osem0, osem1]

        base_chunk = wid * nci
        nvalid = jnp.minimum(nchunk - base_chunk, nci)
        ebase0 = base_chunk * e_chunk
        stage0 = jnp.minimum(ebase0, n * k - nci * e_chunk)
        delta = ebase0 - stage0
        pltpu.async_copy(row_hbm.at[pl.ds(stage0, nci * e_chunk)], idx_ar,
                         osem0)
        pltpu.async_copy(col_hbm.at[pl.ds(stage0, nci * e_chunk)], idx_ac,
                         osem0)
        pltpu.async_copy(x2_hbm, x2_v, osem1)
        pltpu.async_copy(xw_hbm, xw_v, osem1)
        pltpu.async_copy(bvec_hbm, bvec_v, osem1)
        pltpu.make_async_copy(row_hbm.at[pl.ds(stage0, nci * e_chunk)],
                              idx_ar, osem0).wait()
        pltpu.make_async_copy(col_hbm.at[pl.ds(stage0, nci * e_chunk)],
                              idx_ac, osem0).wait()

        def issue(j, b):
            @pl.when(j < nvalid)
            def _():
                off = delta + j * e_chunk
                pltpu.async_copy(
                    x_hbm.at[idx_ar.at[pl.ds(off, e_chunk)]],
                    rows_rs[b], sems[b])
                pltpu.async_copy(
                    x_hbm.at[idx_ac.at[pl.ds(off, e_chunk)]],
                    rows_cs[b], sems[b])

        def compute(j, b):
            rows_r = rows_rs[b]
            rows_c = rows_cs[b]
            acc_v = accs[b]

            @pl.when(j < nvalid)
            def _():
                nbase = (base_chunk + j) * G
                off = delta + j * e_chunk
                pltpu.make_async_copy(
                    x_hbm.at[idx_ar.at[pl.ds(off, e_chunk)]],
                    rows_r, sems[b]).wait()
                pltpu.make_async_copy(
                    x_hbm.at[idx_ac.at[pl.ds(off, e_chunk)]],
                    rows_c, sems[b]).wait()

                @pl.when(j >= 2)
                def _():
                    pltpu.make_async_copy(
                        acc_v, out_hbm.at[pl.ds(nbase, G)], osems[b]).wait()

                @plsc.parallel_loop(0, e_chunk, unroll=2)
                def _(e):
                    acc0 = rows_r[e, pl.ds(0, L)] * rows_c[e, pl.ds(0, L)]
                    acc1 = rows_r[e, pl.ds(L, L)] * rows_c[e, pl.ds(L, L)]
                    for v in range(2, nvr, 2):
                        acc0 = acc0 + (rows_r[e, pl.ds(v * L, L)]
                                       * rows_c[e, pl.ds(v * L, L)])
                        acc1 = acc1 + (rows_r[e, pl.ds((v + 1) * L, L)]
                                       * rows_c[e, pl.ds((v + 1) * L, L)])
                    part_v[pl.ds(e * L, L)] = acc0 + acc1

                @plsc.parallel_loop(0, e_chunk // L, unroll=2)
                def _(gi):
                    base = gi * L
                    evec = (base + lax.iota(jnp.int32, L)) * L
                    terms = [plsc.load_gather(part_v, [evec + j2])
                             for j2 in range(L)]
                    while len(terms) > 1:
                        nxt = [terms[i] + terms[i + 1]
                               for i in range(0, len(terms) - 1, 2)]
                        if len(terms) % 2:
                            nxt.append(terms[-1])
                        terms = nxt
                    pq = terms[0]
                    ir = idx_ar[pl.ds(off + base, L)]
                    ic = idx_ac[pl.ds(off + base, L)]
                    x2r = plsc.load_gather(x2_v, [ir])
                    y2 = plsc.load_gather(x2_v, [ic])
                    pw = plsc.load_gather(xw_v, [ir])
                    qw = plsc.load_gather(xw_v, [ic])
                    den = jnp.maximum(1.0 - 2.0 * pq + x2r * y2, EPS)
                    alpha = -(1.0 - 2.0 * pq + y2) / den
                    beta = (1.0 - x2r) / den
                    sn2 = (alpha * alpha * x2r + 2.0 * alpha * beta * pq
                           + beta * beta * y2 + EPS)
                    gpoly = 1.0 / 15.0
                    for coef in (1.0 / 13.0, 1.0 / 11.0, 1.0 / 9.0, 1.0 / 7.0,
                                 1.0 / 5.0, 1.0 / 3.0, 1.0):
                        gpoly = gpoly * sn2 + coef
                    s = jnp.maximum(1.0 - x2r, EPS) * gpoly
                    logit = s * (alpha * pw + beta * qw) + bvec
                    a = 1.0 / (1.0 + jnp.exp(-logit))
                    u_v[pl.ds(base, L)] = a * s * alpha
                    v_v[pl.ds(base, L)] = a * s * beta

                for nn in range(G):
                    acc = tuple(jnp.zeros((L,), jnp.float32)
                                for _ in range(nvr))
                    for h in range(k // L):
                        ubase = nn * k + h * L
                        uvec = u_v[pl.ds(ubase, L)]
                        vvec = v_v[pl.ds(ubase, L)]

                        @plsc.parallel_loop(0, L, unroll=1, carry=acc)
                        def acc(kk, carry, uvec=uvec, vvec=vvec,
                                ubase=ubase):
                            e = ubase + kk
                            iv = jnp.full((L,), 0, jnp.int32) + kk
                            uu = uvec.at[iv].get(mode="promise_in_bounds")
                            vv = vvec.at[iv].get(mode="promise_in_bounds")
                            return tuple(
                                carry[v] + uu * rows_r[e, pl.ds(v * L, L)]
                                + vv * rows_c[e, pl.ds(v * L, L)]
                                for v in range(nvr))

                    for v in range(nvr):
                        acc_v[nn, pl.ds(v * L, L)] = acc[v]

                pltpu.async_copy(acc_v, out_hbm.at[pl.ds(nbase, G)], osems[b])

        issue(0, 0)
        issue(1, 1)
        pltpu.make_async_copy(x2_hbm, x2_v, osem1).wait()
        pltpu.make_async_copy(xw_hbm, xw_v, osem1).wait()
        pltpu.make_async_copy(bvec_hbm, bvec_v, osem1).wait()
        bvec = bvec_v[...]

        def pair_body(i2, carry):
            for b in range(2):
                j = i2 * 2 + b
                compute(j, b)
                issue(j + 2, b)
            return carry

        lax.fori_loop(0, (nci + 1) // 2, pair_body, 0)

        for b in range(2):
            @pl.when(nvalid > b)
            def _():
                pltpu.make_async_copy(
                    accs[b], out_hbm.at[pl.ds(0, G)], osems[b]).wait()

    return sc_kernel


def _post_body(sup_ref, x_ref, w1_ref, b1_ref, w2_ref, b2_ref, out_ref):
    s = sup_ref[...]
    h1 = jnp.dot(s, w1_ref[...], preferred_element_type=jnp.float32)
    h1 = h1 + b1_ref[...]
    h = h1 / (1.0 + jnp.exp(-h1))
    st = jnp.dot(h, w2_ref[...], preferred_element_type=jnp.float32)
    st = st + b2_ref[...]
    xb = x_ref[...]
    x2 = jnp.sum(xb * xb, axis=-1, keepdims=True)
    un = jnp.sqrt(jnp.sum(st * st, axis=-1, keepdims=True) + EPS)
    lam = 2.0 / jnp.maximum(1.0 - x2, EPS)
    second = jnp.tanh(0.5 * lam * un) * st / un
    s2 = jnp.sum(second * second, axis=-1, keepdims=True)
    xs = jnp.sum(xb * second, axis=-1, keepdims=True)
    num = (1.0 + 2.0 * xs + s2) * xb + (1.0 - x2) * second
    dn = jnp.maximum(1.0 + 2.0 * xs + x2 * s2, EPS)
    o = num / dn
    on = jnp.sqrt(jnp.sum(o * o, axis=-1, keepdims=True) + EPS)
    mx = 1.0 - 1e-3
    out_ref[...] = jnp.where(on > mx, o / on * mx, o)


def _post_call(sup, xf, w1b, b1, w2, b2, blk):
    n, d = xf.shape
    grid = n // blk
    full = lambda i: (0, 0)
    return pl.pallas_call(
        _post_body,
        grid=(grid,),
        in_specs=[
            pl.BlockSpec((blk, d), lambda i: (i, 0)),
            pl.BlockSpec((blk, d), lambda i: (i, 0)),
            pl.BlockSpec((d, d), full),
            pl.BlockSpec((1, d), full),
            pl.BlockSpec((d, d), full),
            pl.BlockSpec((1, d), full),
        ],
        out_specs=pl.BlockSpec((blk, d), lambda i: (i, 0)),
        out_shape=jax.ShapeDtypeStruct((n, d), jnp.float32),
    )(sup, xf, w1b, b1, w2, b2)


def kernel(x, distances, edges, node_mask, edge_mask, att_W, att_b, W1, b1,
           W2, b2):
    b, n, d = x.shape
    k = edges.shape[2]
    xf = x.reshape(n, d)
    wrow = att_W[:d, 0].reshape(1, d)
    x2, xw = _pre_call(xf, wrow, blk=2000)
    rows = edges[0].reshape(n * k)
    cols = edges[1].reshape(n * k)
    bvec = jnp.broadcast_to(att_b.astype(jnp.float32), (L,))
    sup = _make_sc_kernel(n, d, k)(
        xf, rows, cols, x2.reshape(n), xw.reshape(n), bvec)
    out = _post_call(sup, xf, W1[d:], b1.reshape(1, d), W2, b2.reshape(1, d),
                     blk=2000)
    return out.reshape(b, n, d)


# speedup vs baseline: 4.1404x; 1.0030x over previous
import functools
import jax
import jax.numpy as jnp
from jax import lax
from jax.experimental import pallas as pl
from jax.experimental.pallas import tpu as pltpu
from jax.experimental.pallas import tpu_sc as plsc

EPS = 1e-15
L = 16
G = 4

def _pre_body(x_ref, w_ref, x2_ref, xw_ref):
    xb = x_ref[...]
    x2_ref[...] = jnp.sum(xb * xb, axis=-1, keepdims=True)
    xw_ref[...] = jnp.sum(xb * w_ref[...], axis=-1, keepdims=True)


def _pre_call(xf, wrow, blk):
    n, d = xf.shape
    grid = n // blk
    return pl.pallas_call(
        _pre_body,
        grid=(grid,),
        in_specs=[
            pl.BlockSpec((blk, d), lambda i: (i, 0)),
            pl.BlockSpec((1, d), lambda i: (0, 0)),
        ],
        out_specs=[
            pl.BlockSpec((blk, 1), lambda i: (i, 0)),
            pl.BlockSpec((blk, 1), lambda i: (i, 0)),
        ],
        out_shape=[
            jax.ShapeDtypeStruct((n, 1), jnp.float32),
            jax.ShapeDtypeStruct((n, 1), jnp.float32),
        ],
    )(xf, wrow)


def _make_sc_kernel(n, d, k):
    e_chunk = G * k
    nchunk = n // G
    info = plsc.get_sparse_core_info()
    nc, ns = info.num_cores, info.num_subcores
    nw = nc * ns
    nci = (nchunk + nw - 1) // nw
    nvr = d // L

    mesh = plsc.VectorSubcoreMesh(core_axis_name="c", subcore_axis_name="s")

    @functools.partial(
        pl.kernel,
        mesh=mesh,
        compiler_params=pltpu.CompilerParams(needs_layout_passes=False),
        out_type=jax.ShapeDtypeStruct((n, d), jnp.float32),
        scratch_types=[
            pltpu.VMEM((nci * e_chunk,), jnp.int32),
            pltpu.VMEM((nci * e_chunk,), jnp.int32),
            pltpu.VMEM((e_chunk, d), jnp.float32),
            pltpu.VMEM((e_chunk, d), jnp.float32),
            pltpu.VMEM((e_chunk, d), jnp.float32),
            pltpu.VMEM((e_chunk, d), jnp.float32),
            pltpu.VMEM((n,), jnp.float32),
            pltpu.VMEM((n,), jnp.float32),
            pltpu.VMEM((e_chunk * L,), jnp.float32),
            pltpu.VMEM((e_chunk,), jnp.float32),
            pltpu.VMEM((e_chunk,), jnp.float32),
            pltpu.VMEM((L,), jnp.float32),
            pltpu.VMEM((G, d), jnp.float32),
            pltpu.VMEM((G, d), jnp.float32),
            pltpu.SemaphoreType.DMA,
            pltpu.SemaphoreType.DMA,
            pltpu.SemaphoreType.DMA,
            pltpu.SemaphoreType.DMA,
        ],
    )
    def sc_kernel(x_hbm, row_hbm, col_hbm, x2_hbm, xw_hbm, bvec_hbm, out_hbm,
                  idx_ar, idx_ac, rows_r0, rows_c0, rows_r1, rows_c1,
                  x2_v, xw_v, part_v, u_v, v_v, bvec_v, acc0, acc1,
                  sem0, sem1, osem0, osem1):
        wid = lax.axis_index("s") * nc + lax.axis_index("c")
        rows_rs = [rows_r0, rows_r1]
        rows_cs = [rows_c0, rows_c1]
        accs = [acc0, acc1]
        sems = [sem0, sem1]
        osems = [osem0, osem1]

        base_chunk = wid * nci
        nvalid = jnp.minimum(nchunk - base_chunk, nci)
        ebase0 = base_chunk * e_chunk
        stage0 = jnp.minimum(ebase0, n * k - nci * e_chunk)
        delta = ebase0 - stage0
        pltpu.async_copy(row_hbm.at[pl.ds(stage0, nci * e_chunk)], idx_ar,
                         osem0)
        pltpu.async_copy(col_hbm.at[pl.ds(stage0, nci * e_chunk)], idx_ac,
                         osem0)
        pltpu.async_copy(x2_hbm, x2_v, osem1)
        pltpu.async_copy(xw_hbm, xw_v, osem1)
        pltpu.async_copy(bvec_hbm, bvec_v, osem1)
        pltpu.make_async_copy(row_hbm.at[pl.ds(stage0, nci * e_chunk)],
                              idx_ar, osem0).wait()
        pltpu.make_async_copy(col_hbm.at[pl.ds(stage0, nci * e_chunk)],
                              idx_ac, osem0).wait()

        def issue(j, b):
            @pl.when(j < nvalid)
            def _():
                off = delta + j * e_chunk
                pltpu.async_copy(
                    x_hbm.at[idx_ar.at[pl.ds(off, e_chunk)]],
                    rows_rs[b], sems[b])
                pltpu.async_copy(
                    x_hbm.at[idx_ac.at[pl.ds(off, e_chunk)]],
                    rows_cs[b], sems[b])

        def compute(j, b):
            rows_r = rows_rs[b]
            rows_c = rows_cs[b]
            acc_v = accs[b]

            @pl.when(j < nvalid)
            def _():
                nbase = (base_chunk + j) * G
                off = delta + j * e_chunk
                pltpu.make_async_copy(
                    x_hbm.at[idx_ar.at[pl.ds(off, e_chunk)]],
                    rows_r, sems[b]).wait()
                pltpu.make_async_copy(
                    x_hbm.at[idx_ac.at[pl.ds(off, e_chunk)]],
                    rows_c, sems[b]).wait()

                @pl.when(j >= 2)
                def _():
                    pltpu.make_async_copy(
                        acc_v, out_hbm.at[pl.ds(nbase, G)], osems[b]).wait()

                @plsc.parallel_loop(0, e_chunk, unroll=2)
                def _(e):
                    acc0 = rows_r[e, pl.ds(0, L)] * rows_c[e, pl.ds(0, L)]
                    acc1 = rows_r[e, pl.ds(L, L)] * rows_c[e, pl.ds(L, L)]
                    for v in range(2, nvr, 2):
                        acc0 = acc0 + (rows_r[e, pl.ds(v * L, L)]
                                       * rows_c[e, pl.ds(v * L, L)])
                        acc1 = acc1 + (rows_r[e, pl.ds((v + 1) * L, L)]
                                       * rows_c[e, pl.ds((v + 1) * L, L)])
                    part_v[pl.ds(e * L, L)] = acc0 + acc1

                @plsc.parallel_loop(0, e_chunk // L, unroll=1)
                def _(gi):
                    base = gi * L
                    evec = (base + lax.iota(jnp.int32, L)) * L
                    terms = [plsc.load_gather(part_v, [evec + j2])
                             for j2 in range(L)]
                    while len(terms) > 1:
                        nxt = [terms[i] + terms[i + 1]
                               for i in range(0, len(terms) - 1, 2)]
                        if len(terms) % 2:
                            nxt.append(terms[-1])
                        terms = nxt
                    pq = terms[0]
                    ir = idx_ar[pl.ds(off + base, L)]
                    ic = idx_ac[pl.ds(off + base, L)]
                    x2r = plsc.load_gather(x2_v, [ir])
                    y2 = plsc.load_gather(x2_v, [ic])
                    pw = plsc.load_gather(xw_v, [ir])
                    qw = plsc.load_gather(xw_v, [ic])
                    den = jnp.maximum(1.0 - 2.0 * pq + x2r * y2, EPS)
                    alpha = -(1.0 - 2.0 * pq + y2) / den
                    beta = (1.0 - x2r) / den
                    sn2 = (alpha * alpha * x2r + 2.0 * alpha * beta * pq
                           + beta * beta * y2 + EPS)
                    gpoly = 1.0 / 15.0
                    for coef in (1.0 / 13.0, 1.0 / 11.0, 1.0 / 9.0, 1.0 / 7.0,
                                 1.0 / 5.0, 1.0 / 3.0, 1.0):
                        gpoly = gpoly * sn2 + coef
                    s = jnp.maximum(1.0 - x2r, EPS) * gpoly
                    logit = s * (alpha * pw + beta * qw) + bvec
                    a = 1.0 / (1.0 + jnp.exp(-logit))
                    u_v[pl.ds(base, L)] = a * s * alpha
                    v_v[pl.ds(base, L)] = a * s * beta

                for nn in range(G):
                    acc = tuple(jnp.zeros((L,), jnp.float32)
                                for _ in range(nvr))
                    for h in range(k // L):
                        ubase = nn * k + h * L
                        uvec = u_v[pl.ds(ubase, L)]
                        vvec = v_v[pl.ds(ubase, L)]

                        @plsc.parallel_loop(0, L, unroll=1, carry=acc)
                        def acc(kk, carry, uvec=uvec, vvec=vvec,
                                ubase=ubase):
                            e = ubase + kk
                            iv = jnp.full((L,), 0, jnp.int32) + kk
                            uu = uvec.at[iv].get(mode="promise_in_bounds")
                            vv = vvec.at[iv].get(mode="promise_in_bounds")
                            return tuple(
                                carry[v] + uu * rows_r[e, pl.ds(v * L, L)]
                                + vv * rows_c[e, pl.ds(v * L, L)]
                                for v in range(nvr))

                    for v in range(nvr):
                        acc_v[nn, pl.ds(v * L, L)] = acc[v]

                pltpu.async_copy(acc_v, out_hbm.at[pl.ds(nbase, G)], osems[b])

        issue(0, 0)
        issue(1, 1)
        pltpu.make_async_copy(x2_hbm, x2_v, osem1).wait()
        pltpu.make_async_copy(xw_hbm, xw_v, osem1).wait()
        pltpu.make_async_copy(bvec_hbm, bvec_v, osem1).wait()
        bvec = bvec_v[...]

        def pair_body(i2, carry):
            for b in range(2):
                j = i2 * 2 + b
                compute(j, b)
                issue(j + 2, b)
            return carry

        lax.fori_loop(0, (nci + 1) // 2, pair_body, 0)

        for b in range(2):
            @pl.when(nvalid > b)
            def _():
                pltpu.make_async_copy(
                    accs[b], out_hbm.at[pl.ds(0, G)], osems[b]).wait()

    return sc_kernel


def _post_body(sup_ref, x_ref, w1_ref, b1_ref, w2_ref, b2_ref, out_ref):
    s = sup_ref[...]
    h1 = jnp.dot(s, w1_ref[...], preferred_element_type=jnp.float32)
    h1 = h1 + b1_ref[...]
    h = h1 / (1.0 + jnp.exp(-h1))
    st = jnp.dot(h, w2_ref[...], preferred_element_type=jnp.float32)
    st = st + b2_ref[...]
    xb = x_ref[...]
    x2 = jnp.sum(xb * xb, axis=-1, keepdims=True)
    un = jnp.sqrt(jnp.sum(st * st, axis=-1, keepdims=True) + EPS)
    lam = 2.0 / jnp.maximum(1.0 - x2, EPS)
    second = jnp.tanh(0.5 * lam * un) * st / un
    s2 = jnp.sum(second * second, axis=-1, keepdims=True)
    xs = jnp.sum(xb * second, axis=-1, keepdims=True)
    num = (1.0 + 2.0 * xs + s2) * xb + (1.0 - x2) * second
    dn = jnp.maximum(1.0 + 2.0 * xs + x2 * s2, EPS)
    o = num / dn
    on = jnp.sqrt(jnp.sum(o * o, axis=-1, keepdims=True) + EPS)
    mx = 1.0 - 1e-3
    out_ref[...] = jnp.where(on > mx, o / on * mx, o)


def _post_call(sup, xf, w1b, b1, w2, b2, blk):
    n, d = xf.shape
    grid = n // blk
    full = lambda i: (0, 0)
    return pl.pallas_call(
        _post_body,
        grid=(grid,),
        in_specs=[
            pl.BlockSpec((blk, d), lambda i: (i, 0)),
            pl.BlockSpec((blk, d), lambda i: (i, 0)),
            pl.BlockSpec((d, d), full),
            pl.BlockSpec((1, d), full),
            pl.BlockSpec((d, d), full),
            pl.BlockSpec((1, d), full),
        ],
        out_specs=pl.BlockSpec((blk, d), lambda i: (i, 0)),
        out_shape=jax.ShapeDtypeStruct((n, d), jnp.float32),
    )(sup, xf, w1b, b1, w2, b2)


def kernel(x, distances, edges, node_mask, edge_mask, att_W, att_b, W1, b1,
           W2, b2):
    b, n, d = x.shape
    k = edges.shape[2]
    xf = x.reshape(n, d)
    wrow = att_W[:d, 0].reshape(1, d)
    x2, xw = _pre_call(xf, wrow, blk=2000)
    rows = edges[0].reshape(n * k)
    cols = edges[1].reshape(n * k)
    bvec = jnp.broadcast_to(att_b.astype(jnp.float32), (L,))
    sup = _make_sc_kernel(n, d, k)(
        xf, rows, cols, x2.reshape(n), xw.reshape(n), bvec)
    out = _post_call(sup, xf, W1[d:], b1.reshape(1, d), W2, b2.reshape(1, d),
                     blk=2000)
    return out.reshape(b, n, d)


# speedup vs baseline: 4.1438x; 1.0008x over previous
import functools
import jax
import jax.numpy as jnp
from jax import lax
from jax.experimental import pallas as pl
from jax.experimental.pallas import tpu as pltpu
from jax.experimental.pallas import tpu_sc as plsc

EPS = 1e-15
L = 16
G = 4

def _pre_body(x_ref, w_ref, x2_ref, xw_ref):
    xb = x_ref[...]
    x2_ref[...] = jnp.sum(xb * xb, axis=-1, keepdims=True)
    xw_ref[...] = jnp.sum(xb * w_ref[...], axis=-1, keepdims=True)


def _pre_call(xf, wrow, blk):
    n, d = xf.shape
    grid = n // blk
    return pl.pallas_call(
        _pre_body,
        grid=(grid,),
        in_specs=[
            pl.BlockSpec((blk, d), lambda i: (i, 0)),
            pl.BlockSpec((1, d), lambda i: (0, 0)),
        ],
        out_specs=[
            pl.BlockSpec((blk, 1), lambda i: (i, 0)),
            pl.BlockSpec((blk, 1), lambda i: (i, 0)),
        ],
        out_shape=[
            jax.ShapeDtypeStruct((n, 1), jnp.float32),
            jax.ShapeDtypeStruct((n, 1), jnp.float32),
        ],
    )(xf, wrow)


def _make_sc_kernel(n, d, k):
    e_chunk = G * k
    nchunk = n // G
    info = plsc.get_sparse_core_info()
    nc, ns = info.num_cores, info.num_subcores
    nw = nc * ns
    nci = (nchunk + nw - 1) // nw
    nvr = d // L

    mesh = plsc.VectorSubcoreMesh(core_axis_name="c", subcore_axis_name="s")

    @functools.partial(
        pl.kernel,
        mesh=mesh,
        compiler_params=pltpu.CompilerParams(needs_layout_passes=False),
        out_type=jax.ShapeDtypeStruct((n, d), jnp.float32),
        scratch_types=[
            pltpu.VMEM((nci * e_chunk,), jnp.int32),
            pltpu.VMEM((nci * e_chunk,), jnp.int32),
            pltpu.VMEM((e_chunk, d), jnp.float32),
            pltpu.VMEM((e_chunk, d), jnp.float32),
            pltpu.VMEM((e_chunk, d), jnp.float32),
            pltpu.VMEM((e_chunk, d), jnp.float32),
            pltpu.VMEM((n,), jnp.float32),
            pltpu.VMEM((n,), jnp.float32),
            pltpu.VMEM((e_chunk * L,), jnp.float32),
            pltpu.VMEM((e_chunk,), jnp.float32),
            pltpu.VMEM((e_chunk,), jnp.float32),
            pltpu.VMEM((L,), jnp.float32),
            pltpu.VMEM((G, d), jnp.float32),
            pltpu.VMEM((G, d), jnp.float32),
            pltpu.SemaphoreType.DMA,
            pltpu.SemaphoreType.DMA,
            pltpu.SemaphoreType.DMA,
            pltpu.SemaphoreType.DMA,
        ],
    )
    def sc_kernel(x_hbm, row_hbm, col_hbm, x2_hbm, xw_hbm, bvec_hbm, out_hbm,
                  idx_ar, idx_ac, rows_r0, rows_c0, rows_r1, rows_c1,
                  x2_v, xw_v, part_v, u_v, v_v, bvec_v, acc0, acc1,
                  sem0, sem1, osem0, osem1):
        wid = lax.axis_index("s") * nc + lax.axis_index("c")
        rows_rs = [rows_r0, rows_r1]
        rows_cs = [rows_c0, rows_c1]
        accs = [acc0, acc1]
        sems = [sem0, sem1]
        osems = [osem0, osem1]

        base_chunk = wid * nci
        nvalid = jnp.minimum(nchunk - base_chunk, nci)
        ebase0 = base_chunk * e_chunk
        stage0 = jnp.minimum(ebase0, n * k - nci * e_chunk)
        delta = ebase0 - stage0
        pltpu.async_copy(row_hbm.at[pl.ds(stage0, nci * e_chunk)], idx_ar,
                         osem0)
        pltpu.async_copy(col_hbm.at[pl.ds(stage0, nci * e_chunk)], idx_ac,
                         osem0)
        pltpu.async_copy(x2_hbm, x2_v, osem1)
        pltpu.async_copy(xw_hbm, xw_v, osem1)
        pltpu.async_copy(bvec_hbm, bvec_v, osem1)
        pltpu.make_async_copy(row_hbm.at[pl.ds(stage0, nci * e_chunk)],
                              idx_ar, osem0).wait()
        pltpu.make_async_copy(col_hbm.at[pl.ds(stage0, nci * e_chunk)],
                              idx_ac, osem0).wait()

        def issue(j, b):
            @pl.when(j < nvalid)
            def _():
                off = delta + j * e_chunk
                pltpu.async_copy(
                    x_hbm.at[idx_ar.at[pl.ds(off, e_chunk)]],
                    rows_rs[b], sems[b])
                pltpu.async_copy(
                    x_hbm.at[idx_ac.at[pl.ds(off, e_chunk)]],
                    rows_cs[b], sems[b])

        def compute(j, b):
            rows_r = rows_rs[b]
            rows_c = rows_cs[b]
            acc_v = accs[b]

            @pl.when(j < nvalid)
            def _():
                nbase = (base_chunk + j) * G
                off = delta + j * e_chunk
                pltpu.make_async_copy(
                    x_hbm.at[idx_ar.at[pl.ds(off, e_chunk)]],
                    rows_r, sems[b]).wait()
                pltpu.make_async_copy(
                    x_hbm.at[idx_ac.at[pl.ds(off, e_chunk)]],
                    rows_c, sems[b]).wait()

                @pl.when(j >= 2)
                def _():
                    pltpu.make_async_copy(
                        acc_v, out_hbm.at[pl.ds(nbase, G)], osems[b]).wait()

                @plsc.parallel_loop(0, e_chunk, unroll=1)
                def _(e):
                    acc0 = rows_r[e, pl.ds(0, L)] * rows_c[e, pl.ds(0, L)]
                    acc1 = rows_r[e, pl.ds(L, L)] * rows_c[e, pl.ds(L, L)]
                    for v in range(2, nvr, 2):
                        acc0 = acc0 + (rows_r[e, pl.ds(v * L, L)]
                                       * rows_c[e, pl.ds(v * L, L)])
                        acc1 = acc1 + (rows_r[e, pl.ds((v + 1) * L, L)]
                                       * rows_c[e, pl.ds((v + 1) * L, L)])
                    part_v[pl.ds(e * L, L)] = acc0 + acc1

                @plsc.parallel_loop(0, e_chunk // L, unroll=1)
                def _(gi):
                    base = gi * L
                    evec = (base + lax.iota(jnp.int32, L)) * L
                    terms = [plsc.load_gather(part_v, [evec + j2])
                             for j2 in range(L)]
                    while len(terms) > 1:
                        nxt = [terms[i] + terms[i + 1]
                               for i in range(0, len(terms) - 1, 2)]
                        if len(terms) % 2:
                            nxt.append(terms[-1])
                        terms = nxt
                    pq = terms[0]
                    ir = idx_ar[pl.ds(off + base, L)]
                    ic = idx_ac[pl.ds(off + base, L)]
                    x2r = plsc.load_gather(x2_v, [ir])
                    y2 = plsc.load_gather(x2_v, [ic])
                    pw = plsc.load_gather(xw_v, [ir])
                    qw = plsc.load_gather(xw_v, [ic])
                    den = jnp.maximum(1.0 - 2.0 * pq + x2r * y2, EPS)
                    alpha = -(1.0 - 2.0 * pq + y2) / den
                    beta = (1.0 - x2r) / den
                    sn2 = (alpha * alpha * x2r + 2.0 * alpha * beta * pq
                           + beta * beta * y2 + EPS)
                    gpoly = 1.0 / 15.0
                    for coef in (1.0 / 13.0, 1.0 / 11.0, 1.0 / 9.0, 1.0 / 7.0,
                                 1.0 / 5.0, 1.0 / 3.0, 1.0):
                        gpoly = gpoly * sn2 + coef
                    s = jnp.maximum(1.0 - x2r, EPS) * gpoly
                    logit = s * (alpha * pw + beta * qw) + bvec
                    a = 1.0 / (1.0 + jnp.exp(-logit))
                    u_v[pl.ds(base, L)] = a * s * alpha
                    v_v[pl.ds(base, L)] = a * s * beta

                for nn in range(G):
                    acc = tuple(jnp.zeros((L,), jnp.float32)
                                for _ in range(nvr))
                    for h in range(k // L):
                        ubase = nn * k + h * L
                        uvec = u_v[pl.ds(ubase, L)]
                        vvec = v_v[pl.ds(ubase, L)]

                        @plsc.parallel_loop(0, L, unroll=1, carry=acc)
                        def acc(kk, carry, uvec=uvec, vvec=vvec,
                                ubase=ubase):
                            e = ubase + kk
                            iv = jnp.full((L,), 0, jnp.int32) + kk
                            uu = uvec.at[iv].get(mode="promise_in_bounds")
                            vv = vvec.at[iv].get(mode="promise_in_bounds")
                            return tuple(
                                carry[v] + uu * rows_r[e, pl.ds(v * L, L)]
                                + vv * rows_c[e, pl.ds(v * L, L)]
                                for v in range(nvr))

                    for v in range(nvr):
                        acc_v[nn, pl.ds(v * L, L)] = acc[v]

                pltpu.async_copy(acc_v, out_hbm.at[pl.ds(nbase, G)], osems[b])

        issue(0, 0)
        issue(1, 1)
        pltpu.make_async_copy(x2_hbm, x2_v, osem1).wait()
        pltpu.make_async_copy(xw_hbm, xw_v, osem1).wait()
        pltpu.make_async_copy(bvec_hbm, bvec_v, osem1).wait()
        bvec = bvec_v[...]

        def pair_body(i2, carry):
            for b in range(2):
                j = i2 * 2 + b
                compute(j, b)
                issue(j + 2, b)
            return carry

        lax.fori_loop(0, (nci + 1) // 2, pair_body, 0)

        for b in range(2):
            @pl.when(nvalid > b)
            def _():
                pltpu.make_async_copy(
                    accs[b], out_hbm.at[pl.ds(0, G)], osems[b]).wait()

    return sc_kernel


def _post_body(sup_ref, x_ref, w1_ref, b1_ref, w2_ref, b2_ref, out_ref):
    s = sup_ref[...]
    h1 = jnp.dot(s, w1_ref[...], preferred_element_type=jnp.float32)
    h1 = h1 + b1_ref[...]
    h = h1 / (1.0 + jnp.exp(-h1))
    st = jnp.dot(h, w2_ref[...], preferred_element_type=jnp.float32)
    st = st + b2_ref[...]
    xb = x_ref[...]
    x2 = jnp.sum(xb * xb, axis=-1, keepdims=True)
    un = jnp.sqrt(jnp.sum(st * st, axis=-1, keepdims=True) + EPS)
    lam = 2.0 / jnp.maximum(1.0 - x2, EPS)
    second = jnp.tanh(0.5 * lam * un) * st / un
    s2 = jnp.sum(second * second, axis=-1, keepdims=True)
    xs = jnp.sum(xb * second, axis=-1, keepdims=True)
    num = (1.0 + 2.0 * xs + s2) * xb + (1.0 - x2) * second
    dn = jnp.maximum(1.0 + 2.0 * xs + x2 * s2, EPS)
    o = num / dn
    on = jnp.sqrt(jnp.sum(o * o, axis=-1, keepdims=True) + EPS)
    mx = 1.0 - 1e-3
    out_ref[...] = jnp.where(on > mx, o / on * mx, o)


def _post_call(sup, xf, w1b, b1, w2, b2, blk):
    n, d = xf.shape
    grid = n // blk
    full = lambda i: (0, 0)
    return pl.pallas_call(
        _post_body,
        grid=(grid,),
        in_specs=[
            pl.BlockSpec((blk, d), lambda i: (i, 0)),
            pl.BlockSpec((blk, d), lambda i: (i, 0)),
            pl.BlockSpec((d, d), full),
            pl.BlockSpec((1, d), full),
            pl.BlockSpec((d, d), full),
            pl.BlockSpec((1, d), full),
        ],
        out_specs=pl.BlockSpec((blk, d), lambda i: (i, 0)),
        out_shape=jax.ShapeDtypeStruct((n, d), jnp.float32),
    )(sup, xf, w1b, b1, w2, b2)


def kernel(x, distances, edges, node_mask, edge_mask, att_W, att_b, W1, b1,
           W2, b2):
    b, n, d = x.shape
    k = edges.shape[2]
    xf = x.reshape(n, d)
    wrow = att_W[:d, 0].reshape(1, d)
    x2, xw = _pre_call(xf, wrow, blk=2000)
    rows = edges[0].reshape(n * k)
    cols = edges[1].reshape(n * k)
    bvec = jnp.broadcast_to(att_b.astype(jnp.float32), (L,))
    sup = _make_sc_kernel(n, d, k)(
        xf, rows, cols, x2.reshape(n), xw.reshape(n), bvec)
    out = _post_call(sup, xf, W1[d:], b1.reshape(1, d), W2, b2.reshape(1, d),
                     blk=2000)
    return out.reshape(b, n, d)
